# baseline (speedup 1.0000x reference)
import functools

import jax
import jax.numpy as jnp
from jax import lax
from jax.experimental import pallas as pl
from jax.experimental.pallas import tpu as pltpu
from jax.experimental.pallas import tpu_sc as plsc

_IT = False

DN = 128
DE = 16
DG = 128
DH = 128
STEPS = 8
N = 10000
E = 320000

ET = 2000
NT_E = E // ET


def _dot(a, b):
    return jnp.dot(a.astype(jnp.bfloat16), b.astype(jnp.bfloat16),
                   preferred_element_type=jnp.float32)

def _ln(v, s, b):
    m = jnp.mean(v, axis=-1, keepdims=True)
    c = v - m
    var = jnp.mean(c * c, axis=-1, keepdims=True)
    return c / jnp.sqrt(var + 1e-5) * s + b


def _encode_nodes_body(x_ref, nw_ref, nb_ref, wsrc_ref, wdst_ref,
                       h_ref, hs_ref, hd_ref):
    h = jnp.maximum(
        _dot(x_ref[...], nw_ref[...])
        + nb_ref[...], 0.0)
    h_ref[...] = h
    hs_ref[...] = _dot(h, wsrc_ref[...])
    hd_ref[...] = _dot(h, wdst_ref[...])


def _encode_nodes(x, nw, nb, wsrc, wdst):
    return pl.pallas_call(
        _encode_nodes_body,
        out_shape=[
            jax.ShapeDtypeStruct((N, DN), jnp.float32),
            jax.ShapeDtypeStruct((N, DN), jnp.float32),
            jax.ShapeDtypeStruct((N, DN), jnp.float32),
        ],
        interpret=_IT,
    )(x, nw, nb, wsrc, wdst)


def _encode_edges_body(ea_ref, ew_ref, eb_ref, e_ref):
    e_ref[...] = jnp.maximum(
        _dot(ea_ref[...], ew_ref[...])
        + eb_ref[...], 0.0)


def _encode_edges(edge_attr, ew, eb):
    return pl.pallas_call(
        _encode_edges_body,
        grid=(NT_E,),
        in_specs=[
            pl.BlockSpec((ET, DE), lambda t: (t, 0)),
            pl.BlockSpec((DE, DE), lambda t: (0, 0)),
            pl.BlockSpec((1, DE), lambda t: (0, 0)),
        ],
        out_specs=pl.BlockSpec((ET, DE), lambda t: (t, 0)),
        out_shape=jax.ShapeDtypeStruct((E, DE), jnp.float32),
        interpret=_IT,
    )(edge_attr, ew, eb)


def _edge_mlp_body(gsum_ref, e_ref, we_ref, w2_ref, cvec_ref, eb2_ref,
                   els_ref, elb_ref, eout_ref, esum_ref):
    t = pl.program_id(0)
    pre = (gsum_ref[...]
           + _dot(e_ref[...], we_ref[...])
           + cvec_ref[...])
    hid = jnp.maximum(pre, 0.0)
    out = _dot(hid, w2_ref[...]) + eb2_ref[...]
    enew = _ln(out, els_ref[...], elb_ref[...])
    eout_ref[...] = enew

    @pl.when(t == 0)
    def _init():
        esum_ref[...] = jnp.zeros_like(esum_ref)

    esum_ref[...] += jnp.sum(enew, axis=0, keepdims=True)


def _edge_mlp(gsum, e, we, w2, cvec, eb2, els, elb):
    return pl.pallas_call(
        _edge_mlp_body,
        grid=(NT_E,),
        in_specs=[
            pl.BlockSpec((ET, DN), lambda t: (t, 0)),
            pl.BlockSpec((ET, DE), lambda t: (t, 0)),
            pl.BlockSpec((DE, DN), lambda t: (0, 0)),
            pl.BlockSpec((DH, DE), lambda t: (0, 0)),
            pl.BlockSpec((1, DN), lambda t: (0, 0)),
            pl.BlockSpec((1, DE), lambda t: (0, 0)),
            pl.BlockSpec((1, DE), lambda t: (0, 0)),
            pl.BlockSpec((1, DE), lambda t: (0, 0)),
        ],
        out_specs=[
            pl.BlockSpec((ET, DE), lambda t: (t, 0)),
            pl.BlockSpec((1, DE), lambda t: (0, 0)),
        ],
        out_shape=[
            jax.ShapeDtypeStruct((E, DE), jnp.float32),
            jax.ShapeDtypeStruct((1, DE), jnp.float32),
        ],
        interpret=_IT,
    )(gsum, e, we, w2, cvec, eb2, els, elb)


def _node_update_body(h_ref, agg_ref, g_ref, esum_ref,
                      wnh_ref, wnagg_ref, wng_ref, nb1_ref, nw2_ref, nb2_ref,
                      nls_ref, nlb_ref,
                      wgh_ref, wge_ref, wgg_ref, gb1_ref, gw2_ref, gb2_ref,
                      gls_ref, glb_ref,
                      wsrc_ref, wdst_ref, wgedge_ref, eb1_ref,
                      hnew_ref, gnew_ref, hs_ref, hd_ref, cvec_ref):
    h = h_ref[...]
    agg = agg_ref[0] + agg_ref[1]
    g = g_ref[...]
    cn = (_dot(g, wng_ref[...])
          + nb1_ref[...])
    pre = (_dot(h, wnh_ref[...])
           + _dot(agg, wnagg_ref[...])
           + cn)
    hid = jnp.maximum(pre, 0.0)
    hnew = _ln(_dot(hid, nw2_ref[...])
               + nb2_ref[...], nls_ref[...], nlb_ref[...])
    hnew_ref[...] = hnew

    mean_h = jnp.mean(hnew, axis=0, keepdims=True)
    mean_e = esum_ref[...] * (1.0 / E)
    gpre = (_dot(mean_h, wgh_ref[...])
            + _dot(mean_e, wge_ref[...])
            + _dot(g, wgg_ref[...])
            + gb1_ref[...])
    ghid = jnp.maximum(gpre, 0.0)
    gnew = _ln(_dot(ghid, gw2_ref[...])
               + gb2_ref[...], gls_ref[...], glb_ref[...])
    gnew_ref[...] = gnew

    hs_ref[...] = _dot(hnew, wsrc_ref[...])
    hd_ref[...] = _dot(hnew, wdst_ref[...])
    cvec_ref[...] = _dot(gnew, wgedge_ref[...]) + eb1_ref[...]


def _node_update(h, agg, g, esum, wnh, wnagg, wng, nb1, nw2, nb2, nls, nlb,
                 wgh, wge, wgg, gb1, gw2, gb2, gls, glb,
                 wsrc, wdst, wgedge, eb1):
    return pl.pallas_call(
        _node_update_body,
        out_shape=[
            jax.ShapeDtypeStruct((N, DN), jnp.float32),
            jax.ShapeDtypeStruct((1, DG), jnp.float32),
            jax.ShapeDtypeStruct((N, DN), jnp.float32),
            jax.ShapeDtypeStruct((N, DN), jnp.float32),
            jax.ShapeDtypeStruct((1, DN), jnp.float32),
        ],
        interpret=_IT,
    )(h, agg, g, esum, wnh, wnagg, wng, nb1, nw2, nb2, nls, nlb,
      wgh, wge, wgg, gb1, gw2, gb2, gls, glb, wsrc, wdst, wgedge, eb1)


def _decode_body(h_ref, mask_ref, w1_ref, b1_ref, w2_ref, b2_ref, out_ref):
    hid = jnp.maximum(
        _dot(h_ref[...], w1_ref[...])
        + b1_ref[...], 0.0)
    per_node = (_dot(hid, w2_ref[...])
                + b2_ref[...])
    out_ref[...] = jnp.sum(per_node * mask_ref[...], axis=0, keepdims=True)


def _decode(h, mask_f, w1, b1, w2, b2):
    return pl.pallas_call(
        _decode_body,
        out_shape=jax.ShapeDtypeStruct((1, 1), jnp.float32),
        interpret=_IT,
    )(h, mask_f, w1, b1, w2, b2)


NC = 2
NS = 16
NW = NC * NS
EPW = E // NW
CH = 80
NCHUNK = EPW // CH
NPS = N // NS

@functools.cache
def _sc_mesh():
    return plsc.VectorSubcoreMesh(
        core_axis_name="c", subcore_axis_name="s",
        num_cores=NC, num_subcores=NS)


def _gather_body(hs_hbm, hd_hbm, src_hbm, dst_hbm, out_hbm,
                 isrc, idst, ra0, ra1, rb0, rb1,
                 sa0, sa1, sb0, sb1, so0, so1):
    wid = lax.axis_index("s") * NC + lax.axis_index("c")
    base = wid * EPW
    ra, rb = (ra0, ra1), (rb0, rb1)
    sa, sb, so = (sa0, sa1), (sb0, sb1), (so0, so1)

    pltpu.sync_copy(src_hbm.at[wid], isrc)
    pltpu.sync_copy(dst_hbm.at[wid], idst)

    def start_gathers(t, p):
        pltpu.async_copy(hs_hbm.at[isrc.at[t]], ra[p], sa[p])
        pltpu.async_copy(hd_hbm.at[idst.at[t]], rb[p], sb[p])

    start_gathers(0, 0)

    def turn(t, p):
        pltpu.make_async_copy(hs_hbm.at[isrc.at[t]], ra[p], sa[p]).wait()
        pltpu.make_async_copy(hd_hbm.at[idst.at[t]], rb[p], sb[p]).wait()

        q = 1 - p

        @pl.when(t + 1 < NCHUNK)
        def _prefetch():
            @pl.when(t >= 1)
            def _wait_store():
                pltpu.make_async_copy(
                    ra[q], out_hbm.at[pl.ds(base, CH)], so[q]).wait()

            start_gathers(t + 1, q)

        def add_row(r, carry):
            for grp in range(DN // 16):
                sl = pl.ds(grp * 16, 16)
                plsc.addupdate(ra[p].at[r, sl], rb[p][r, sl])
            return carry

        lax.fori_loop(0, CH, add_row, 0)
        pltpu.async_copy(ra[p], out_hbm.at[pl.ds(base + t * CH, CH)], so[p])

    def pair(t2, carry):
        turn(2 * t2, 0)

        @pl.when(2 * t2 + 1 < NCHUNK)
        def _odd():
            turn(2 * t2 + 1, 1)

        return carry

    lax.fori_loop(0, (NCHUNK + 1) // 2, pair, 0)

    for p in (1, 0):
        pltpu.make_async_copy(ra[p], out_hbm.at[pl.ds(base, CH)],
                              so[p]).wait()


@functools.cache
def _gather_sum_sc():
    return pl.kernel(
        _gather_body,
        out_type=jax.ShapeDtypeStruct((E, DN), jnp.float32),
        mesh=_sc_mesh(),
        scratch_types=(
            [pltpu.VMEM((NCHUNK, CH), jnp.int32)] * 2
            + [pltpu.VMEM((CH, DN), jnp.float32)] * 4
            + [pltpu.SemaphoreType.DMA] * 6
        ),
    )


def _gather_sum(hs, hd, src, dst):
    return _gather_sum_sc()(hs, hd, src.reshape(NW, NCHUNK, CH),
                            dst.reshape(NW, NCHUNK, CH))


ZR = 624
ZTAIL = N - NS * ZR


def _scatter_body(e_hbm, dst_hbm, zin_hbm, out_hbm, idx_v, rows_v, shared, sem):
    cid = lax.axis_index("c")
    sid = lax.axis_index("s")

    pltpu.sync_copy(zin_hbm.at[pl.ds(sid * ZR, ZR)],
                    shared.at[pl.ds(sid * ZR, ZR)])

    @pl.when(sid == 0)
    def _ztail():
        pltpu.sync_copy(zin_hbm.at[pl.ds(NS * ZR, ZTAIL)],
                        shared.at[pl.ds(NS * ZR, ZTAIL)])

    plsc.subcore_barrier()

    wid = sid * NC + cid
    pltpu.sync_copy(dst_hbm.at[wid], idx_v)

    def do_half(lo, cnt):
        pltpu.sync_copy(e_hbm.at[wid, pl.ds(lo, cnt)],
                        rows_v.at[pl.ds(0, cnt)])

        def fire(j, carry):
            pltpu.async_copy(rows_v.at[j], shared.at[idx_v.at[lo + j]],
                             sem, add=True)
            return carry

        lax.fori_loop(0, cnt, fire, 0)

        def drain(j, carry):
            pltpu.make_async_copy(rows_v.at[0], shared.at[idx_v.at[0]],
                                  sem).wait()
            return carry

        lax.fori_loop(0, cnt, drain, 0)

    half = (NCHUNK + 1) // 2
    do_half(0, half)
    do_half(half, NCHUNK - half)
    plsc.subcore_barrier()
    pltpu.sync_copy(shared.at[pl.ds(sid * ZR, ZR)],
                    out_hbm.at[cid, pl.ds(sid * ZR, ZR)])

    @pl.when(sid == 0)
    def _dtail():
        pltpu.sync_copy(shared.at[pl.ds(NS * ZR, ZTAIL)],
                        out_hbm.at[cid, pl.ds(NS * ZR, ZTAIL)])


@functools.cache
def _scatter_agg_sc():
    return pl.kernel(
        _scatter_body,
        out_type=jax.ShapeDtypeStruct((NC, N, DE), jnp.float32),
        mesh=_sc_mesh(),
        scratch_types=[
            pltpu.VMEM((NCHUNK, CH), jnp.int32),
            pltpu.VMEM(((NCHUNK + 1) // 2, CH, DE), jnp.float32),
            pltpu.VMEM_SHARED((N, DE), jnp.float32),
            pltpu.SemaphoreType.DMA,
        ],
        compiler_params=pltpu.CompilerParams(use_tc_tiling_on_sc=False),
    )


def _scatter_agg(e, dst):
    return _scatter_agg_sc()(e.reshape(NW, NCHUNK, CH, DE),
                             dst.reshape(NW, NCHUNK, CH),
                             jnp.zeros((N, DE), jnp.float32))


def kernel(x, edge_index, edge_attr, instruction_mask, task_id, params):
    p = params
    src = edge_index[0]
    dst = edge_index[1]

    e_w1 = p["e_w1"]
    wsrc, wdst, we, wgedge = (e_w1[:DN], e_w1[DN:2 * DN],
                              e_w1[2 * DN:2 * DN + DE], e_w1[2 * DN + DE:])
    n_w1 = p["n_w1"]
    wnh, wnagg, wng = n_w1[:DN], n_w1[DN:DN + DE], n_w1[DN + DE:]
    g_w1 = p["g_w1"]
    wgh, wge, wgg = g_w1[:DN], g_w1[DN:DN + DE], g_w1[DN + DE:]

    row = lambda v: v.reshape(1, -1)

    h, hs, hd = _encode_nodes(x, p["enc_nw"], row(p["enc_nb"]), wsrc, wdst)
    e = _encode_edges(edge_attr, p["enc_ew"], row(p["enc_eb"]))
    g = jnp.zeros((1, DG), jnp.float32)
    cvec = row(p["e_b1"])

    for _ in range(STEPS):
        gsum = _gather_sum(hs, hd, src, dst)
        e, esum = _edge_mlp(gsum, e, we, p["e_w2"], cvec, row(p["e_b2"]),
                            row(p["e_ls"]), row(p["e_lb"]))
        agg = _scatter_agg(e, dst)
        h, g, hs, hd, cvec = _node_update(
            h, agg, g, esum, wnh, wnagg, wng, row(p["n_b1"]), p["n_w2"],
            row(p["n_b2"]),
            row(p["n_ls"]), row(p["n_lb"]),
            wgh, wge, wgg, row(p["g_b1"]), p["g_w2"], row(p["g_b2"]),
            row(p["g_ls"]), row(p["g_lb"]),
            wsrc, wdst, wgedge, row(p["e_b1"]))

    mask_f = instruction_mask.astype(jnp.float32).reshape(N, 1)
    out = _decode(h, mask_f, p["d_w1"], row(p["d_b1"]), p["d_w2"],
                  row(p["d_b2"]))
    return out[0, 0]
